# baseline (speedup 1.0000x reference)
import functools

import jax
import jax.numpy as jnp
from jax import lax
from jax.experimental import pallas as pl
from jax.experimental.pallas import tpu as pltpu
from jax.experimental.pallas import tpu_sc as plsc

B = 16384
D = 32
F = 16
NC = 2
NS = 16
NW = NC * NS
RPW = B // NW
IDX_CHUNK = 128
N_IDX_CHUNKS = RPW // IDX_CHUNK
ROWS_PER_STEP = 16
N_STEPS = RPW // ROWS_PER_STEP


def _sc_body(uidx_hbm, midx_hbm, feat_hbm, utab_hbm, mtab_hbm, w_hbm,
             out_hbm, uidx_v, midx_v, feat_v, urows_v, mrows_v, w_v,
             out_v, sem):
    wid = lax.axis_index("s") * NC + lax.axis_index("c")
    base = wid * RPW

    pltpu.sync_copy(uidx_hbm.at[pl.ds(base, RPW)], uidx_v)
    pltpu.sync_copy(midx_hbm.at[pl.ds(base, RPW)], midx_v)
    pltpu.sync_copy(feat_hbm.at[pl.ds(base, RPW), :], feat_v)
    pltpu.sync_copy(w_hbm, w_v)

    descs = []
    for j in range(N_IDX_CHUNKS):
        sl = pl.ds(j * IDX_CHUNK, IDX_CHUNK)
        descs.append(pltpu.async_copy(
            utab_hbm.at[uidx_v.at[sl]], urows_v.at[sl, :], sem))
        descs.append(pltpu.async_copy(
            mtab_hbm.at[midx_v.at[sl]], mrows_v.at[sl, :], sem))
    for d in descs:
        d.wait()

    wv0 = w_v[pl.ds(0, 16)]
    wv1 = w_v[pl.ds(16, 16)]
    wvf = w_v[pl.ds(32, 16)]
    wvb = w_v[pl.ds(48, 16)]
    w_emb = [wv0[d] for d in range(16)] + [wv1[d] for d in range(16)]
    w_feat = [wvf[f] for f in range(F)]
    bias = wvb[0]
    lanes = lax.iota(jnp.int32, 16)

    def step(c, carry):
        rows = c * ROWS_PER_STEP + lanes
        acc = jnp.zeros((16,), jnp.float32)
        for d in range(D):
            col = jnp.full((16,), d, jnp.int32)
            gu = plsc.load_gather(urows_v, [rows, col])
            gm = plsc.load_gather(mrows_v, [rows, col])
            acc = acc + gu * gm * w_emb[d]
        uvals = uidx_v[pl.ds(c * ROWS_PER_STEP, ROWS_PER_STEP)]
        mvals = midx_v[pl.ds(c * ROWS_PER_STEP, ROWS_PER_STEP)]
        valid = (uvals != 0) & (mvals != 0)
        acc = jnp.where(valid, acc, 0.0)
        for f in range(F):
            col = jnp.full((16,), f, jnp.int32)
            gf = plsc.load_gather(feat_v, [rows, col])
            acc = acc + gf * w_feat[f]
        out_v[pl.ds(c * ROWS_PER_STEP, ROWS_PER_STEP)] = acc + bias
        return carry

    lax.fori_loop(0, N_STEPS, step, 0, unroll=False)
    pltpu.sync_copy(out_v, out_hbm.at[pl.ds(base, RPW)])


@jax.jit
def kernel(user_idx, movie_idx, features, user_table, movie_table, fc_w, fc_b):
    w_all = jnp.concatenate([
        jnp.reshape(fc_w, (-1,)).astype(jnp.float32),
        jnp.reshape(fc_b, (-1,)).astype(jnp.float32),
        jnp.zeros((64 - D - F - 1,), jnp.float32),
    ])
    mesh = plsc.VectorSubcoreMesh(core_axis_name="c", subcore_axis_name="s",
                                  num_cores=NC, num_subcores=NS)
    fn = pl.kernel(
        _sc_body,
        out_type=jax.ShapeDtypeStruct((B,), jnp.float32),
        mesh=mesh,
        scratch_types=[
            pltpu.VMEM((RPW,), jnp.int32),
            pltpu.VMEM((RPW,), jnp.int32),
            pltpu.VMEM((RPW, F), jnp.float32),
            pltpu.VMEM((RPW, D), jnp.float32),
            pltpu.VMEM((RPW, D), jnp.float32),
            pltpu.VMEM((64,), jnp.float32),
            pltpu.VMEM((RPW,), jnp.float32),
            pltpu.SemaphoreType.DMA,
        ],
        compiler_params=pltpu.CompilerParams(needs_layout_passes=False,
                                             use_tc_tiling_on_sc=False),
    )
    return fn(user_idx, movie_idx, features, user_table, movie_table, w_all)
